# baseline (speedup 1.0000x reference)
import functools

import jax
import jax.numpy as jnp
from jax import lax
from jax.experimental import pallas as pl
from jax.experimental.pallas import tpu as pltpu
from jax.experimental.pallas import tpu_sc as plsc


def _encoder_sc(context, A_tables, C_last, B, M, S, V, E):
    info = plsc.get_sparse_core_info()
    NC = info.num_cores
    NW = NC * info.num_subcores
    BW = B // NW
    NG = (M + 15) // 16
    assert B % NW == 0 and E == 32
    inv_e = 1.0 / E
    ACW = 16 * NG + 1

    mesh = plsc.VectorSubcoreMesh(core_axis_name="c", subcore_axis_name="s")

    @functools.partial(
        pl.kernel,
        mesh=mesh,
        out_type=jax.ShapeDtypeStruct((B, M, E), jnp.float32),
        compiler_params=pltpu.CompilerParams(use_tc_tiling_on_sc=False,
                                             needs_layout_passes=False),
        scratch_types=[
            pltpu.VMEM((M + 1, S), jnp.int32),
            pltpu.VMEM((1008,), jnp.int32),
            pltpu.VMEM((1008,), jnp.int32),
            pltpu.VMEM((M * S, E), jnp.float32),
            pltpu.VMEM((M * S, E), jnp.float32),
            pltpu.VMEM((E, ACW), jnp.float32),
            pltpu.VMEM((E, ACW), jnp.float32),
            pltpu.VMEM((E, ACW), jnp.float32),
            pltpu.VMEM((E * 16,), jnp.float32),
            pltpu.VMEM((16 * NG, E + 1), jnp.float32),
            pltpu.VMEM((M, E), jnp.float32),
            pltpu.SemaphoreType.DMA,
        ],
    )
    def enc(ctx_hbm, at_hbm, cl_hbm, o_hbm,
            idx2_v, idx_a, idx_b, rows_a, rows_b, g1_t, g2_t, gc_t, es_v,
            op_v, out_v, sem):
        wid = lax.axis_index("s") * NC + lax.axis_index("c")
        b0 = wid * BW
        lanes = lax.iota(jnp.int32, 16)
        a1_hbm = at_hbm.at[1]
        a2_hbm = at_hbm.at[2]

        NFL = (M * S + 15) // 16
        BOFF = list(range(0, M * S, 128))
        BSZ = [min(128, M * S - o) for o in BOFF]

        def stage_flatten(b, idx_v):
            pltpu.sync_copy(ctx_hbm.at[b], idx2_v.at[pl.ds(0, M)])

            def fl(g, carry):
                flat = g * 16 + lanes
                r = flat // S
                c = flat - r * S
                idx_v[pl.ds(g * 16, 16)] = plsc.load_gather(idx2_v, [r, c])
                return carry

            lax.fori_loop(0, NFL, fl, 0, unroll=4)

        def fire(t_hbm, idx_v, rows_v):
            return [pltpu.async_copy(t_hbm.at[idx_v.at[pl.ds(o, n)]],
                                     rows_v.at[pl.ds(o, n)], sem)
                    for o, n in zip(BOFF, BSZ)]

        def wait_fired(t_hbm, idx_v, rows_v):
            for o, n in zip(BOFF, BSZ):
                pltpu.make_async_copy(t_hbm.at[idx_v.at[pl.ds(o, n)]],
                                      rows_v.at[pl.ds(o, n)],
                                      sem).wait()

        def reduce_rows(rows_v, acc_t):
            def pair_body(p, carry):
                r0 = p * S
                lo = rows_v[r0, pl.ds(0, 16)]
                hi = rows_v[r0, pl.ds(16, 16)]
                for s in range(1, S):
                    lo = lo + rows_v[r0 + s, pl.ds(0, 16)]
                    hi = hi + rows_v[r0 + s, pl.ds(16, 16)]
                cols = jnp.full((16,), p, jnp.int32)
                plsc.store_scatter(acc_t, [lanes, cols], lo)
                plsc.store_scatter(acc_t, [lanes + 16, cols], hi)
                return carry

            lax.fori_loop(0, M, pair_body, 0, unroll=False)

        def epilogue_store(b):
            for g in range(NG):
                g16 = g * 16

                def p2(e, s1):
                    v1 = g1_t[e, pl.ds(g16, 16)]
                    e1 = jnp.exp(v1 * v1 * inv_e)
                    es_v[pl.ds(e * 16, 16)] = e1
                    return s1 + e1

                s1 = lax.fori_loop(0, E, p2, jnp.zeros((16,), jnp.float32),
                                   unroll=8)
                r1 = 1.0 / s1

                def p3(e, m2):
                    v1 = g1_t[e, pl.ds(g16, 16)]
                    v2 = g2_t[e, pl.ds(g16, 16)]
                    a1 = es_v[pl.ds(e * 16, 16)] * r1
                    t2 = v2 * (v1 * inv_e + v2 * a1)
                    es_v[pl.ds(e * 16, 16)] = t2
                    return jnp.maximum(m2, t2)

                m2 = lax.fori_loop(0, E, p3,
                                   jnp.full((16,), -jnp.inf, jnp.float32),
                                   unroll=8)

                def p4(e, s2):
                    e2 = jnp.exp(es_v[pl.ds(e * 16, 16)] - m2)
                    es_v[pl.ds(e * 16, 16)] = e2
                    return s2 + e2

                s2 = lax.fori_loop(0, E, p4, jnp.zeros((16,), jnp.float32),
                                   unroll=8)
                r2 = 1.0 / s2

                if g16 + 16 <= M:
                    msk = None
                else:
                    msk = g16 + lanes < M

                def p5(e, carry2):
                    vc = gc_t[e, pl.ds(g16, 16)]
                    o = vc * es_v[pl.ds(e * 16, 16)] * r2
                    cols = jnp.full((16,), e, jnp.int32)
                    if msk is None:
                        plsc.store_scatter(op_v, [g16 + lanes, cols], o)
                    else:
                        plsc.store_scatter(op_v, [g16 + lanes, cols], o,
                                           mask=msk)
                    return carry2

                lax.fori_loop(0, E, p5, 0, unroll=8)

            def compact(p, carry3):
                out_v[p, pl.ds(0, 16)] = op_v[p, pl.ds(0, 16)]
                out_v[p, pl.ds(16, 16)] = op_v[p, pl.ds(16, 16)]
                return carry3

            lax.fori_loop(0, M, compact, 0, unroll=False)
            pltpu.sync_copy(out_v, o_hbm.at[b])

        def half_body(b, idx_cur, idx_nxt, rows_cur, rows_nxt):
            wait_fired(a1_hbm, idx_cur, rows_cur)
            c2 = fire(a2_hbm, idx_cur, rows_nxt)
            reduce_rows(rows_cur, g1_t)
            for cp in c2:
                cp.wait()
            c3 = fire(cl_hbm, idx_cur, rows_cur)
            reduce_rows(rows_nxt, g2_t)
            for cp in c3:
                cp.wait()
            b_nxt = jnp.minimum(b + 1, B - 1)
            stage_flatten(b_nxt, idx_nxt)
            fire(a1_hbm, idx_nxt, rows_nxt)
            reduce_rows(rows_cur, gc_t)
            epilogue_store(b)

        stage_flatten(b0, idx_a)
        fire(a1_hbm, idx_a, rows_a)

        def chunk_pair(i, carry):
            b = b0 + 2 * i
            half_body(b, idx_a, idx_b, rows_a, rows_b)
            half_body(b + 1, idx_b, idx_a, rows_b, rows_a)
            return carry

        lax.fori_loop(0, BW // 2, chunk_pair, 0, unroll=False)
        wait_fired(a1_hbm, idx_a, rows_a)

    return enc(context, A_tables, C_last)


def kernel(context, A_tables, C_last):
    B, M, S = context.shape
    hops, V, E = A_tables.shape
    assert hops == 3 and E == 32 and B % 64 == 0
    return _encoder_sc(context, A_tables, C_last, B, M, S, V, E)


# speedup vs baseline: 1.0671x; 1.0671x over previous
import functools

import jax
import jax.numpy as jnp
from jax import lax
from jax.experimental import pallas as pl
from jax.experimental.pallas import tpu as pltpu
from jax.experimental.pallas import tpu_sc as plsc


def _encoder_sc(context, A_tables, C_last, B, M, S, V, E):
    info = plsc.get_sparse_core_info()
    NC = info.num_cores
    NW = NC * info.num_subcores
    BW = B // NW
    NG = (M + 15) // 16
    assert B % NW == 0 and E == 32
    inv_e = 1.0 / E
    ACW = 16 * NG + 1

    mesh = plsc.VectorSubcoreMesh(core_axis_name="c", subcore_axis_name="s")

    @functools.partial(
        pl.kernel,
        mesh=mesh,
        out_type=jax.ShapeDtypeStruct((B, M, E), jnp.float32),
        compiler_params=pltpu.CompilerParams(use_tc_tiling_on_sc=False,
                                             needs_layout_passes=False),
        scratch_types=[
            pltpu.VMEM((8, M * S // 8), jnp.int32),
            pltpu.VMEM((8, M * S // 8), jnp.int32),
            pltpu.VMEM((M * S, E), jnp.float32),
            pltpu.VMEM((M * S, E), jnp.float32),
            pltpu.VMEM((M * S, E), jnp.float32),
            pltpu.VMEM((E, ACW), jnp.float32),
            pltpu.VMEM((E, ACW), jnp.float32),
            pltpu.VMEM((E, ACW), jnp.float32),
            pltpu.VMEM((E * 16,), jnp.float32),
            pltpu.VMEM((16 * NG, E + 1), jnp.float32),
            pltpu.VMEM((M, E), jnp.float32),
            pltpu.SemaphoreType.DMA,
            pltpu.SemaphoreType.DMA,
            pltpu.SemaphoreType.DMA,
        ],
    )
    def enc(ctx_hbm, at_hbm, cl_hbm, o_hbm,
            idx_a, idx_b, rows_a, rows_b, rows_c, g1_t, g2_t, gc_t, es_v,
            op_v, out_v, sem1, sem2, sem3):
        wid = lax.axis_index("s") * NC + lax.axis_index("c")
        b0 = wid * BW
        lanes = lax.iota(jnp.int32, 16)
        a1_hbm = at_hbm.at[1]
        a2_hbm = at_hbm.at[2]

        RB = M * S // 8

        def fire(t_hbm, idx_v, rows_v, sem):
            return [pltpu.async_copy(t_hbm.at[idx_v.at[j]],
                                     rows_v.at[pl.ds(j * RB, RB)], sem)
                    for j in range(8)]

        def wait_fired(t_hbm, idx_v, rows_v, sem):
            for j in range(8):
                pltpu.make_async_copy(t_hbm.at[idx_v.at[j]],
                                      rows_v.at[pl.ds(j * RB, RB)],
                                      sem).wait()

        def reduce_rows(rows_v, acc_t):
            def pair_body(p, carry):
                r0 = p * S
                lo = rows_v[r0, pl.ds(0, 16)]
                hi = rows_v[r0, pl.ds(16, 16)]
                for s in range(1, S):
                    lo = lo + rows_v[r0 + s, pl.ds(0, 16)]
                    hi = hi + rows_v[r0 + s, pl.ds(16, 16)]
                cols = jnp.full((16,), p, jnp.int32)
                plsc.store_scatter(acc_t, [lanes, cols], lo)
                plsc.store_scatter(acc_t, [lanes + 16, cols], hi)
                return carry

            lax.fori_loop(0, M, pair_body, 0, unroll=False)

        def epilogue_store(b):
            for g in range(NG):
                g16 = g * 16

                def p2(e, s1):
                    v1 = g1_t[e, pl.ds(g16, 16)]
                    e1 = jnp.exp(v1 * v1 * inv_e)
                    es_v[pl.ds(e * 16, 16)] = e1
                    return s1 + e1

                s1 = lax.fori_loop(0, E, p2, jnp.zeros((16,), jnp.float32),
                                   unroll=8)
                r1 = 1.0 / s1

                def p3(e, m2):
                    v1 = g1_t[e, pl.ds(g16, 16)]
                    v2 = g2_t[e, pl.ds(g16, 16)]
                    a1 = es_v[pl.ds(e * 16, 16)] * r1
                    t2 = v2 * (v1 * inv_e + v2 * a1)
                    es_v[pl.ds(e * 16, 16)] = t2
                    return jnp.maximum(m2, t2)

                m2 = lax.fori_loop(0, E, p3,
                                   jnp.full((16,), -jnp.inf, jnp.float32),
                                   unroll=8)

                def p4(e, s2):
                    e2 = jnp.exp(es_v[pl.ds(e * 16, 16)] - m2)
                    es_v[pl.ds(e * 16, 16)] = e2
                    return s2 + e2

                s2 = lax.fori_loop(0, E, p4, jnp.zeros((16,), jnp.float32),
                                   unroll=8)
                r2 = 1.0 / s2

                if g16 + 16 <= M:
                    msk = None
                else:
                    msk = g16 + lanes < M

                def p5(e, carry2):
                    vc = gc_t[e, pl.ds(g16, 16)]
                    o = vc * es_v[pl.ds(e * 16, 16)] * r2
                    cols = jnp.full((16,), e, jnp.int32)
                    if msk is None:
                        plsc.store_scatter(op_v, [g16 + lanes, cols], o)
                    else:
                        plsc.store_scatter(op_v, [g16 + lanes, cols], o,
                                           mask=msk)
                    return carry2

                lax.fori_loop(0, E, p5, 0, unroll=8)

            def compact(p, carry3):
                out_v[p, pl.ds(0, 16)] = op_v[p, pl.ds(0, 16)]
                out_v[p, pl.ds(16, 16)] = op_v[p, pl.ds(16, 16)]
                return carry3

            lax.fori_loop(0, M, compact, 0, unroll=False)
            pltpu.sync_copy(out_v, o_hbm.at[b])

        def half_body(b, idx_cur, idx_nxt):
            c2 = fire(a2_hbm, idx_cur, rows_b, sem2)
            c3 = fire(cl_hbm, idx_cur, rows_c, sem3)
            wait_fired(a1_hbm, idx_cur, rows_a, sem1)
            reduce_rows(rows_a, g1_t)
            for cp in c2:
                cp.wait()
            reduce_rows(rows_b, g2_t)
            b_nxt = jnp.minimum(b + 1, B - 1)
            pltpu.sync_copy(ctx_hbm.at[b_nxt], idx_nxt)
            fire(a1_hbm, idx_nxt, rows_a, sem1)
            for cp in c3:
                cp.wait()
            reduce_rows(rows_c, gc_t)
            epilogue_store(b)

        pltpu.sync_copy(ctx_hbm.at[b0], idx_a)
        fire(a1_hbm, idx_a, rows_a, sem1)

        def chunk_pair(i, carry):
            b = b0 + 2 * i
            half_body(b, idx_a, idx_b)
            half_body(b + 1, idx_b, idx_a)
            return carry

        lax.fori_loop(0, BW // 2, chunk_pair, 0, unroll=False)
        wait_fired(a1_hbm, idx_a, rows_a, sem1)

    ctx3 = context.reshape(B, 8, M * S // 8)
    return enc(ctx3, A_tables, C_last)


def kernel(context, A_tables, C_last):
    B, M, S = context.shape
    hops, V, E = A_tables.shape
    assert hops == 3 and E == 32 and B % 64 == 0
    return _encoder_sc(context, A_tables, C_last, B, M, S, V, E)


# speedup vs baseline: 1.1143x; 1.0442x over previous
import functools

import jax
import jax.numpy as jnp
from jax import lax
from jax.experimental import pallas as pl
from jax.experimental.pallas import tpu as pltpu
from jax.experimental.pallas import tpu_sc as plsc


def _encoder_sc(context, A_tables, C_last, B, M, S, V, E):
    info = plsc.get_sparse_core_info()
    NC = info.num_cores
    NW = NC * info.num_subcores
    BW = B // NW
    NG = (M + 15) // 16
    assert B % NW == 0 and E == 32
    inv_e = 1.0 / E
    ACW = 16 * NG + 1

    mesh = plsc.VectorSubcoreMesh(core_axis_name="c", subcore_axis_name="s")

    @functools.partial(
        pl.kernel,
        mesh=mesh,
        out_type=jax.ShapeDtypeStruct((B, M, E), jnp.float32),
        compiler_params=pltpu.CompilerParams(use_tc_tiling_on_sc=False,
                                             needs_layout_passes=False),
        scratch_types=[
            pltpu.VMEM((8, M * S // 8), jnp.int32),
            pltpu.VMEM((8, M * S // 8), jnp.int32),
            pltpu.VMEM((M * S, E), jnp.float32),
            pltpu.VMEM((M * S, E), jnp.float32),
            pltpu.VMEM((E, ACW), jnp.float32),
            pltpu.VMEM((E, ACW), jnp.float32),
            pltpu.VMEM((E, ACW), jnp.float32),
            pltpu.VMEM((E * 16,), jnp.float32),
            pltpu.VMEM((16 * NG, E + 1), jnp.float32),
            pltpu.VMEM((M, E), jnp.float32),
            pltpu.SemaphoreType.DMA,
        ],
    )
    def enc(ctx_hbm, at_hbm, cl_hbm, o_hbm,
            idx_a, idx_b, rows_a, rows_b, g1_t, g2_t, gc_t, es_v, op_v,
            out_v, sem):
        wid = lax.axis_index("s") * NC + lax.axis_index("c")
        b0 = wid * BW
        lanes = lax.iota(jnp.int32, 16)
        a1_hbm = at_hbm.at[1]
        a2_hbm = at_hbm.at[2]

        RB = M * S // 8

        def fire(t_hbm, idx_v, rows_v):
            return [pltpu.async_copy(t_hbm.at[idx_v.at[j]],
                                     rows_v.at[pl.ds(j * RB, RB)], sem)
                    for j in range(8)]

        def wait_fired(t_hbm, idx_v, rows_v):
            for j in range(8):
                pltpu.make_async_copy(t_hbm.at[idx_v.at[j]],
                                      rows_v.at[pl.ds(j * RB, RB)],
                                      sem).wait()

        def reduce_rows(rows_v, acc_t):
            def pair_body(p, carry):
                r0 = p * S
                lo = rows_v[r0, pl.ds(0, 16)]
                hi = rows_v[r0, pl.ds(16, 16)]
                for s in range(1, S):
                    lo = lo + rows_v[r0 + s, pl.ds(0, 16)]
                    hi = hi + rows_v[r0 + s, pl.ds(16, 16)]
                cols = jnp.full((16,), p, jnp.int32)
                plsc.store_scatter(acc_t, [lanes, cols], lo)
                plsc.store_scatter(acc_t, [lanes + 16, cols], hi)
                return carry

            lax.fori_loop(0, M, pair_body, 0, unroll=False)

        def epilogue_store(b):
            for g in range(NG):
                g16 = g * 16

                def p2(e, s1):
                    v1 = g1_t[e, pl.ds(g16, 16)]
                    e1 = jnp.exp(v1 * v1 * inv_e)
                    es_v[pl.ds(e * 16, 16)] = e1
                    return s1 + e1

                s1 = lax.fori_loop(0, E, p2, jnp.zeros((16,), jnp.float32),
                                   unroll=8)
                r1 = 1.0 / s1

                def p3(e, m2):
                    v1 = g1_t[e, pl.ds(g16, 16)]
                    v2 = g2_t[e, pl.ds(g16, 16)]
                    a1 = es_v[pl.ds(e * 16, 16)] * r1
                    t2 = v2 * (v1 * inv_e + v2 * a1)
                    es_v[pl.ds(e * 16, 16)] = t2
                    return jnp.maximum(m2, t2)

                m2 = lax.fori_loop(0, E, p3,
                                   jnp.full((16,), -jnp.inf, jnp.float32),
                                   unroll=8)

                def p4(e, s2):
                    e2 = jnp.exp(es_v[pl.ds(e * 16, 16)] - m2)
                    es_v[pl.ds(e * 16, 16)] = e2
                    return s2 + e2

                s2 = lax.fori_loop(0, E, p4, jnp.zeros((16,), jnp.float32),
                                   unroll=8)
                r2 = 1.0 / s2

                if g16 + 16 <= M:
                    msk = None
                else:
                    msk = g16 + lanes < M

                def p5(e, carry2):
                    vc = gc_t[e, pl.ds(g16, 16)]
                    o = vc * es_v[pl.ds(e * 16, 16)] * r2
                    cols = jnp.full((16,), e, jnp.int32)
                    if msk is None:
                        plsc.store_scatter(op_v, [g16 + lanes, cols], o)
                    else:
                        plsc.store_scatter(op_v, [g16 + lanes, cols], o,
                                           mask=msk)
                    return carry2

                lax.fori_loop(0, E, p5, 0, unroll=8)

            def compact(p, carry3):
                out_v[p, pl.ds(0, 16)] = op_v[p, pl.ds(0, 16)]
                out_v[p, pl.ds(16, 16)] = op_v[p, pl.ds(16, 16)]
                return carry3

            lax.fori_loop(0, M, compact, 0, unroll=False)
            pltpu.sync_copy(out_v, o_hbm.at[b])

        def half_body(b, idx_cur, idx_nxt, rows_cur, rows_nxt):
            wait_fired(a1_hbm, idx_cur, rows_cur)
            c2 = fire(a2_hbm, idx_cur, rows_nxt)
            reduce_rows(rows_cur, g1_t)
            for cp in c2:
                cp.wait()
            c3 = fire(cl_hbm, idx_cur, rows_cur)
            reduce_rows(rows_nxt, g2_t)
            for cp in c3:
                cp.wait()
            b_nxt = jnp.minimum(b + 1, B - 1)
            pltpu.sync_copy(ctx_hbm.at[b_nxt], idx_nxt)
            fire(a1_hbm, idx_nxt, rows_nxt)
            reduce_rows(rows_cur, gc_t)
            epilogue_store(b)

        pltpu.sync_copy(ctx_hbm.at[b0], idx_a)
        fire(a1_hbm, idx_a, rows_a)

        def chunk_pair(i, carry):
            b = b0 + 2 * i
            half_body(b, idx_a, idx_b, rows_a, rows_b)
            half_body(b + 1, idx_b, idx_a, rows_b, rows_a)
            return carry

        lax.fori_loop(0, BW // 2, chunk_pair, 0, unroll=False)
        wait_fired(a1_hbm, idx_a, rows_a)

    ctx3 = context.reshape(B, 8, M * S // 8)
    return enc(ctx3, A_tables, C_last)


def kernel(context, A_tables, C_last):
    B, M, S = context.shape
    hops, V, E = A_tables.shape
    assert hops == 3 and E == 32 and B % 64 == 0
    return _encoder_sc(context, A_tables, C_last, B, M, S, V, E)
